# baseline (speedup 1.0000x reference)
import functools

import jax
import jax.numpy as jnp
from jax import lax
from jax.experimental import pallas as pl
from jax.experimental.pallas import tpu as pltpu
from jax.experimental.pallas import tpu_sc as plsc

B = 1024
NUM = 13
NCAT = 26
V = 1000
D = 16
R1 = 832


def _stage1_body(x_ref, w_ref, idx_ref, wfm_ref):
    x = x_ref[...]
    pos = lax.broadcasted_iota(jnp.int32, (1, V), 1).astype(jnp.float32)
    lab = jnp.sum(x * pos, axis=1, keepdims=True)
    f = lax.broadcasted_iota(jnp.int32, (R1, 1), 0) % NCAT
    idx_ref[...] = (lab + 0.5).astype(jnp.int32) + f * V
    wfm_ref[...] = jnp.sum(x * w_ref[...], axis=1, keepdims=True)


def _stage1(x2, w_tiled):
    n = B * NCAT
    return pl.pallas_call(
        _stage1_body,
        grid=(n // R1,),
        in_specs=[
            pl.BlockSpec((R1, V), lambda i: (i, 0)),
            pl.BlockSpec((R1, V), lambda i: (0, 0)),
        ],
        out_specs=[
            pl.BlockSpec((R1, 1), lambda i: (i, 0)),
            pl.BlockSpec((R1, 1), lambda i: (i, 0)),
        ],
        out_shape=[
            jax.ShapeDtypeStruct((n, 1), jnp.int32),
            jax.ShapeDtypeStruct((n, 1), jnp.float32),
        ],
    )(x2, w_tiled)


def _sc_gather(table, idx):
    info = plsc.get_sparse_core_info()
    nw = info.num_cores * info.num_subcores
    n = idx.shape[0]
    b_per_w = n // nw
    mesh = plsc.VectorSubcoreMesh(core_axis_name="c", subcore_axis_name="s")

    @functools.partial(
        pl.kernel, mesh=mesh,
        compiler_params=pltpu.CompilerParams(use_tc_tiling_on_sc=False),
        out_type=jax.ShapeDtypeStruct((n, D), jnp.float32),
        scratch_types=[
            pltpu.VMEM((b_per_w,), jnp.int32),
            pltpu.VMEM((b_per_w, D), jnp.float32),
            pltpu.SemaphoreType.DMA,
        ],
    )
    def k(table_hbm, idx_hbm, out_hbm, idx_v, rows_v, sem):
        wid = lax.axis_index("s") * info.num_cores + lax.axis_index("c")
        base = wid * b_per_w
        pltpu.sync_copy(idx_hbm.at[pl.ds(base, b_per_w)], idx_v)
        pltpu.async_copy(table_hbm.at[idx_v], rows_v, sem).wait()
        pltpu.sync_copy(rows_v, out_hbm.at[pl.ds(base, b_per_w)])

    return k(table, idx)


def _stage3_body(num_ref, cat_ref, wfmv_ref, wnum_ref, ncst_ref, s_ref, a_ref,
                 w1c_ref, b1_ref, w2_ref, b2_ref, w3_ref, b3_ref, wfm_ref,
                 bfm_ref, out_ref):
    hp = lax.Precision.HIGHEST
    num = num_ref[...]
    cat = cat_ref[...]
    nsum = jnp.dot(num, wnum_ref[...], precision=hp) + ncst_ref[...]
    csum = jnp.dot(cat, s_ref[...], precision=hp)
    yfme = jnp.sum(nsum * csum, axis=1, keepdims=True)
    yfms = (jnp.dot(num, wfm_ref[...], precision=hp)
            + jnp.sum(wfmv_ref[...], axis=1, keepdims=True) + bfm_ref[...])
    h1 = jnp.maximum(
        jnp.dot(num, a_ref[...], precision=hp)
        + jnp.dot(cat, w1c_ref[...], precision=hp) + b1_ref[...], 0.0)
    h2 = jnp.maximum(jnp.dot(h1, w2_ref[...], precision=hp) + b2_ref[...], 0.0)
    yd = jnp.dot(h2, w3_ref[...], precision=hp) + b3_ref[...]
    out_ref[...] = jax.nn.sigmoid(yfme + yfms + yd)


def _stage3(num, catf, wfmv, wnum, ncst, s, a, w1c, b1c, w2, b2, w3, b3,
            wfm13, bfm):
    return pl.pallas_call(
        _stage3_body,
        out_shape=jax.ShapeDtypeStruct((B, 1), jnp.float32),
    )(num, catf, wfmv, wnum, ncst, s, a, w1c, b1c, w2, b2, w3, b3,
      wfm13, bfm)


def kernel(numeric_feats, categorical_feats, W_num, b_num, emb_tables,
           W_fm, b_fm, W1, b1, W2, b2, W3, b3):
    f32 = jnp.float32
    wfm_rows = W_fm[NUM:, 0].reshape(NCAT, V)
    w_tiled = jnp.tile(wfm_rows, (R1 // NCAT, 1))

    a = jnp.einsum('id,idm->im', W_num, W1[:NUM * D].reshape(NUM, D, -1))
    b1c = (b1 + b_num.reshape(-1) @ W1[:NUM * D])[None]
    ncst = b_num.sum(0)[None]
    s = jnp.tile(jnp.eye(D, dtype=f32), (NCAT, 1))

    x2 = categorical_feats.reshape(B * NCAT, V)
    idx2, wfmv = _stage1(x2, w_tiled)

    cat_rows = _sc_gather(emb_tables.reshape(NCAT * V, D), idx2.reshape(-1))
    catf = cat_rows.reshape(B, NCAT * D)

    return _stage3(numeric_feats, catf, wfmv.reshape(B, NCAT),
                   W_num, ncst, s, a,
                   W1[NUM * D:], b1c, W2, b2[None], W3, b3[None],
                   W_fm[:NUM], b_fm[None])
